# Initial kernel scaffold; baseline (speedup 1.0000x reference)
import jax
import jax.numpy as jnp
from jax.experimental import pallas as pl


def kernel(pos, x, batch, ew, eb, w1, b1, w2, b2, ln_g, ln_b):
    raise NotImplementedError("write your pallas kernel here")


# speedup vs baseline: 4.7395x; 4.7395x over previous
import functools
import numpy as np
import jax
import jax.numpy as jnp
from jax import lax
from jax.experimental import pallas as pl
from jax.experimental.pallas import tpu as pltpu
from jax.experimental.pallas import tpu_sc as plsc

_B = 4
_N_ATOMS = 8192
_N_TYPES = 16
_GRID_SIZE = 8
_N_GRID = _GRID_SIZE ** 3
_NG_TOT = _B * _N_GRID
_CD = 128
_L = 4
_K_G = 32
_K_A = 8
_CUTOFF = 5.0
_SPACING = 1.5
_NGAUSS = 20
_N_NODES = _N_ATOMS + _NG_TOT
_E_A = _N_ATOMS * _K_A
_E_G = _NG_TOT * _K_G
_E_TOT = _E_A + _E_G

_off_np = (np.exp(np.linspace(np.log(1.0), np.log(_CUTOFF + 1.0), _NGAUSS,
                              dtype=np.float32)) - 1.0).astype(np.float32)
_diff_np = np.diff(_off_np)
_diff_np = np.concatenate([_diff_np[:1], _diff_np])
_coeff_np = (-0.5 / _diff_np ** 2).astype(np.float32)

_half = (_GRID_SIZE - 1) * _SPACING / 2.0
_g1 = np.linspace(-_half, _half, _GRID_SIZE, dtype=np.float32)
_mesh = np.meshgrid(_g1, _g1, _g1, indexing='ij')
_gc_np = np.stack(_mesh, axis=-1).reshape(-1, 3).astype(np.float32)


def _knn_body(k, exclude_self, qb_rows, bf16_dot, q_ref, kt_ref, idx_ref,
              dist_ref):
    i = pl.program_id(0)
    q = q_ref[...]
    qx = q[:, 0:1]
    qy = q[:, 1:2]
    qz = q[:, 2:3]
    qb = q[:, 3:4]
    kx = kt_ref[0:1, :]
    ky = kt_ref[1:2, :]
    kz = kt_ref[2:3, :]
    kb = kt_ref[3:4, :]
    qq = qx * qx + qy * qy + qz * qz
    kk = kx * kx + ky * ky + kz * kz

    if bf16_dot:
        def bf(v):
            return v.astype(jnp.bfloat16).astype(jnp.float32)
        dot = (bf(qx) * bf(kx) + bf(qy) * bf(ky)) + bf(qz) * bf(kz)
    else:
        dot = (qx * kx + qy * ky) + qz * kz
    d2 = qq + kk - 2.0 * dot
    d2 = jnp.where(qb != kb, 1e10, d2)
    nk = kt_ref.shape[1]
    col = lax.broadcasted_iota(jnp.int32, (qb_rows, nk), 1)
    if exclude_self:
        qidx = i * qb_rows + lax.broadcasted_iota(jnp.int32, (qb_rows, nk), 0)
        d2 = jnp.where(col == qidx, 1e10, d2)
    idx_cols = []
    dist_cols = []
    big = jnp.int32(2 ** 30)
    for _ in range(k):
        m = jnp.min(d2, axis=1, keepdims=True)
        sel = jnp.where(d2 <= m, col, big)
        a = jnp.min(sel, axis=1, keepdims=True)
        idx_cols.append(a)
        hit = col == a
        kxs = jnp.sum(jnp.where(hit, kx, 0.0), axis=1, keepdims=True)
        kys = jnp.sum(jnp.where(hit, ky, 0.0), axis=1, keepdims=True)
        kzs = jnp.sum(jnp.where(hit, kz, 0.0), axis=1, keepdims=True)
        dx = kxs - qx
        dy = kys - qy
        dz = kzs - qz
        dist_cols.append(jnp.sqrt((dx * dx + dy * dy) + dz * dz + 1e-12))
        d2 = jnp.where(hit, 1e10, d2)
    idx_ref[...] = jnp.concatenate(idx_cols, axis=1)
    dist_ref[...] = jnp.concatenate(dist_cols, axis=1)


def _knn(qpacked, ktpacked, k, exclude_self, qb_rows, bf16_dot):
    nq = qpacked.shape[0]
    nk = ktpacked.shape[1]
    grid = (nq // qb_rows,)
    return pl.pallas_call(
        functools.partial(_knn_body, k, exclude_self, qb_rows, bf16_dot),
        grid=grid,
        in_specs=[
            pl.BlockSpec((qb_rows, 8), lambda i: (i, 0)),
            pl.BlockSpec((8, nk), lambda i: (0, 0)),
        ],
        out_specs=[
            pl.BlockSpec((qb_rows, k), lambda i: (i, 0)),
            pl.BlockSpec((qb_rows, k), lambda i: (i, 0)),
        ],
        out_shape=[
            jax.ShapeDtypeStruct((nq, k), jnp.int32),
            jax.ShapeDtypeStruct((nq, k), jnp.float32),
        ],
    )(qpacked, ktpacked)


_SC_CHUNK = 512
_SC_NW = 32


def _sc_gather_impl(table, idx2d):
    nchunks = idx2d.shape[0]
    per_w = nchunks // _SC_NW
    mesh = plsc.VectorSubcoreMesh(core_axis_name="c", subcore_axis_name="s")

    @functools.partial(
        pl.kernel, mesh=mesh,
        out_type=jax.ShapeDtypeStruct((nchunks * _SC_CHUNK, _CD), jnp.float32),
        scratch_types=[
            pltpu.VMEM((_SC_CHUNK,), jnp.int32),
            pltpu.VMEM((_SC_CHUNK, _CD), jnp.float32),
            pltpu.SemaphoreType.DMA,
        ],
    )
    def k(table_hbm, idx_hbm, out_hbm, idx_v, rows_v, sem):
        cid = lax.axis_index("c")
        sid = lax.axis_index("s")
        wid = sid * 2 + cid
        for c in range(per_w):
            chunk = wid * per_w + c
            pltpu.sync_copy(idx_hbm.at[chunk], idx_v)
            pltpu.async_copy(table_hbm.at[idx_v], rows_v, sem).wait()
            pltpu.sync_copy(rows_v, out_hbm.at[pl.ds(chunk * _SC_CHUNK, _SC_CHUNK)])

    return k(table, idx2d)


def _gather_rows(table, idx2d):
    return _sc_gather_impl(table, idx2d)


def _prep_body(h_ref, w1a_ref, w1b_ref, ha_ref, hc_ref):
    h = h_ref[...]
    ha_ref[...] = jnp.dot(h, w1a_ref[...], preferred_element_type=jnp.float32)
    hc_ref[...] = jnp.dot(h, w1b_ref[...], preferred_element_type=jnp.float32)


def _prep(h, w1a, w1b):
    nb = 1024
    return pl.pallas_call(
        _prep_body,
        grid=(_N_NODES // nb,),
        in_specs=[
            pl.BlockSpec((nb, _CD), lambda i: (i, 0)),
            pl.BlockSpec((_CD, _CD), lambda i: (0, 0)),
            pl.BlockSpec((_CD, _CD), lambda i: (0, 0)),
        ],
        out_specs=[
            pl.BlockSpec((nb, _CD), lambda i: (i, 0)),
            pl.BlockSpec((nb, _CD), lambda i: (i, 0)),
        ],
        out_shape=[
            jax.ShapeDtypeStruct((_N_NODES, _CD), jnp.float32),
            jax.ShapeDtypeStruct((_N_NODES, _CD), jnp.float32),
        ],
    )(h, w1a, w1b)


def _edge_body(k, db, g_ref, hc_ref, dist_ref, ew_ref, w1c_ref, eb_ref,
               b1_ref, smc_ref, out_ref):
    w1c = w1c_ref[...]
    ew2 = jnp.dot(ew_ref[...], w1c, preferred_element_type=jnp.float32)
    btot = b1_ref[...] + jnp.dot(eb_ref[...], w1c,
                                 preferred_element_type=jnp.float32)
    hc = hc_ref[...]
    off = smc_ref[0:1, :]
    coeff = smc_ref[1:2, :]
    acc = jnp.zeros((db, _CD), jnp.float32)
    for j in range(k):
        d_j = dist_ref[:, j:j + 1]
        d_j = jnp.clip(d_j, 0.0, _CUTOFF)
        sm = jnp.exp(coeff * (d_j - off) ** 2)
        ef = jnp.dot(sm, ew2, preferred_element_type=jnp.float32)
        pre = g_ref[j, :, :] + hc + ef + btot
        acc = acc + jnp.maximum(pre, 0.0)
    out_ref[...] = acc


def _edge(g3, hc, dist, ew_l, w1c, eb_l, b1_l, smc, k, db, hc_row_off):
    nd = g3.shape[1]
    nblocks = nd // db
    hc_blk_off = hc_row_off // db
    return pl.pallas_call(
        functools.partial(_edge_body, k, db),
        grid=(nblocks,),
        in_specs=[
            pl.BlockSpec((k, db, _CD), lambda i: (0, i, 0)),
            pl.BlockSpec((db, _CD), lambda i: (i + hc_blk_off, 0)),
            pl.BlockSpec((db, k), lambda i: (i, 0)),
            pl.BlockSpec((_NGAUSS, _CD), lambda i: (0, 0)),
            pl.BlockSpec((_CD, _CD), lambda i: (0, 0)),
            pl.BlockSpec((1, _CD), lambda i: (0, 0)),
            pl.BlockSpec((1, _CD), lambda i: (0, 0)),
            pl.BlockSpec((2, _NGAUSS), lambda i: (0, 0)),
        ],
        out_specs=pl.BlockSpec((db, _CD), lambda i: (i, 0)),
        out_shape=jax.ShapeDtypeStruct((nd, _CD), jnp.float32),
    )(g3, hc, dist, ew_l, w1c, eb_l, b1_l, smc)


def _node_body(nb, h_ref, s_ref, w2_ref, b2_ref, lng_ref, lnb_ref, out_ref):
    pid = pl.program_id(0)
    inv = jnp.where(pid < _N_ATOMS // nb, 1.0 / _K_A, 1.0 / _K_G)
    mean_q = s_ref[...] * inv
    upd = jnp.dot(mean_q, w2_ref[...],
                  preferred_element_type=jnp.float32) + b2_ref[...]
    h2 = h_ref[...] + upd
    mu = jnp.mean(h2, axis=1, keepdims=True)
    xc = h2 - mu
    var = jnp.mean(xc * xc, axis=1, keepdims=True)
    out_ref[...] = xc / jnp.sqrt(var + 1e-5) * lng_ref[...] + lnb_ref[...]


def _node(h, s, w2_l, b2_l, lng_l, lnb_l):
    nb = 1024
    return pl.pallas_call(
        functools.partial(_node_body, nb),
        grid=(_N_NODES // nb,),
        in_specs=[
            pl.BlockSpec((nb, _CD), lambda i: (i, 0)),
            pl.BlockSpec((nb, _CD), lambda i: (i, 0)),
            pl.BlockSpec((_CD, _CD), lambda i: (0, 0)),
            pl.BlockSpec((1, _CD), lambda i: (0, 0)),
            pl.BlockSpec((1, _CD), lambda i: (0, 0)),
            pl.BlockSpec((1, _CD), lambda i: (0, 0)),
        ],
        out_specs=pl.BlockSpec((nb, _CD), lambda i: (i, 0)),
        out_shape=jax.ShapeDtypeStruct((_N_NODES, _CD), jnp.float32),
    )(h, s, w2_l, b2_l, lng_l, lnb_l)


def kernel(pos, x, batch, ew, eb, w1, b1, w2, b2, ln_g, ln_b):
    batch_f = batch.astype(jnp.float32)
    q_atoms = jnp.concatenate(
        [pos, batch_f[:, None], jnp.zeros((_N_ATOMS, 4), jnp.float32)],
        axis=1)
    kt = jnp.concatenate(
        [pos.T, batch_f[None, :], jnp.zeros((4, _N_ATOMS), jnp.float32)],
        axis=0)

    gc = jnp.asarray(_gc_np)
    grid_pos = jnp.tile(gc, (_B, 1))
    grid_batch_f = jnp.repeat(jnp.arange(_B, dtype=jnp.float32), _N_GRID)
    q_grid = jnp.concatenate(
        [grid_pos, grid_batch_f[:, None], jnp.zeros((_NG_TOT, 4), jnp.float32)],
        axis=1)

    idx_a, dist_a = _knn(q_atoms, kt, _K_A, True, 128, True)
    idx_g, dist_g = _knn(q_grid, kt, _K_G, False, 128, True)

    row_all = jnp.concatenate(
        [idx_a.T.reshape(-1), idx_g.T.reshape(-1)], axis=0)
    idx2d = row_all.reshape(-1, _SC_CHUNK)

    feat = (x[:, None] == jnp.arange(_N_TYPES)[None, :]).astype(jnp.float32)
    h = jnp.concatenate(
        [jnp.concatenate(
            [feat, jnp.zeros((_N_ATOMS, _CD - _N_TYPES), jnp.float32)], 1),
         jnp.zeros((_NG_TOT, _CD), jnp.float32)], axis=0)

    smc = jnp.stack([jnp.asarray(_off_np), jnp.asarray(_coeff_np)], axis=0)

    for l in range(_L):
        w1a = w1[l, :_CD]
        w1b = w1[l, _CD:2 * _CD]
        w1c = w1[l, 2 * _CD:]
        ha, hc = _prep(h, w1a, w1b)
        g = _gather_rows(ha, idx2d)
        g_a = g[:_E_A].reshape(_K_A, _N_ATOMS, _CD)
        g_g = g[_E_A:].reshape(_K_G, _NG_TOT, _CD)
        s_a = _edge(g_a, hc, dist_a, ew[l], w1c, eb[l].reshape(1, -1),
                    b1[l].reshape(1, -1), smc, _K_A, 512, 0)
        s_g = _edge(g_g, hc, dist_g, ew[l], w1c, eb[l].reshape(1, -1),
                    b1[l].reshape(1, -1), smc, _K_G, 256, _N_ATOMS)
        s = jnp.concatenate([s_a, s_g], axis=0)
        h = _node(h, s, w2[l], b2[l].reshape(1, -1),
                  ln_g[l].reshape(1, -1), ln_b[l].reshape(1, -1))

    return h[_N_ATOMS:].reshape(_B, _N_GRID, _CD)
